# baseline (speedup 1.0000x reference)
import functools

import jax
import jax.numpy as jnp
from jax import lax
from jax.experimental import pallas as pl
from jax.experimental.pallas import tpu as pltpu
from jax.experimental.pallas import tpu_sc as plsc

DIM = 64
BATCH = 16384

_NC, _NS = 2, 16
_NW = _NC * _NS
_BPW = BATCH // _NW
_CHUNK = 128
_NCHUNK = _BPW // _CHUNK


@functools.partial(
    pl.kernel,
    out_type=jax.ShapeDtypeStruct((_NW * _NCHUNK, _CHUNK, DIM), jnp.float32),
    mesh=plsc.VectorSubcoreMesh(core_axis_name="c", subcore_axis_name="s"),
    scratch_types=[
        pltpu.VMEM((_NCHUNK, _CHUNK), jnp.int32),
        pltpu.VMEM((_NCHUNK, _CHUNK, DIM), jnp.float32),
        pltpu.SemaphoreType.DMA,
    ],
    compiler_params=pltpu.CompilerParams(use_tc_tiling_on_sc=False),
)
def _gather(table_hbm, nodes_hbm, out_hbm, idx_v, rows_v, sem):
    wid = lax.axis_index("s") * _NC + lax.axis_index("c")
    base = wid * _NCHUNK
    pltpu.sync_copy(nodes_hbm.at[pl.ds(base, _NCHUNK)], idx_v)
    copies = [
        pltpu.async_copy(table_hbm.at[idx_v.at[j]], rows_v.at[j], sem)
        for j in range(_NCHUNK)
    ]
    for c in copies:
        c.wait()
    pltpu.sync_copy(rows_v, out_hbm.at[pl.ds(base, _NCHUNK)])


def kernel(features, nodes):
    nodes2d = nodes.reshape(_NW * _NCHUNK, _CHUNK)
    out = _gather(features, nodes2d)
    return out.reshape(BATCH, DIM)


# speedup vs baseline: 2.3537x; 2.3537x over previous
import functools

import jax
import jax.numpy as jnp
from jax import lax
from jax.experimental import pallas as pl
from jax.experimental.pallas import tpu as pltpu
from jax.experimental.pallas import tpu_sc as plsc

DIM = 64
BATCH = 16384

_NC, _NS = 2, 16
_NW = _NC * _NS
_BPW = BATCH // _NW
_GRP = 128
_NGRP = _BPW // _GRP
_RING = 4


@functools.partial(
    pl.kernel,
    out_type=jax.ShapeDtypeStruct((DIM, BATCH), jnp.float32),
    mesh=plsc.VectorSubcoreMesh(core_axis_name="c", subcore_axis_name="s"),
    scratch_types=[
        pltpu.VMEM((_BPW,), jnp.int32),
        pltpu.VMEM((2, _RING, DIM, 128), jnp.float32),
        pltpu.VMEM((DIM, _GRP), jnp.float32),
        pltpu.SemaphoreType.DMA,
        pltpu.SemaphoreType.DMA,
    ],
    compiler_params=pltpu.CompilerParams(needs_layout_passes=False),
)
def _gather_t(table_t, nodes_hbm, out_t, nodes_s, blocks_v, group_v, sem0, sem1):
    wid = lax.axis_index("s") * _NC + lax.axis_index("c")
    base = wid * _BPW
    pltpu.sync_copy(nodes_hbm.at[pl.ds(base, _BPW)], nodes_s)
    sems = (sem0, sem1)
    iota16 = lax.iota(jnp.int32, 16)

    for g in range(_NGRP):
        g0 = g * _GRP

        def quad(q, _):
            t16 = g0 + q * 16
            ndv = nodes_s[pl.ds(t16, 16)]
            jbase = q * 16

            def fire(b, parity):
                sem = sems[parity]
                for r in range(_RING):
                    n = ndv[_RING * b + r]
                    col = pl.multiple_of((n >> 7) * 128, 128)
                    pltpu.async_copy(
                        table_t.at[:, pl.ds(col, 128)],
                        blocks_v.at[parity, r],
                        sem,
                    )

            def drain(parity):
                sem = sems[parity]
                for r in range(_RING):
                    pltpu.make_async_copy(
                        table_t.at[:, pl.ds(0, 128)],
                        blocks_v.at[parity, r],
                        sem,
                    ).wait()

            def extract(b, parity):
                for r in range(_RING):
                    n = ndv[_RING * b + r]
                    c = jnp.full((16,), n & 127, dtype=jnp.int32)
                    j = jnp.full((16,), jbase + _RING * b + r, dtype=jnp.int32)
                    blk = blocks_v.at[parity, r]
                    for k in range(DIM // 16):
                        fk = iota16 + (k * 16)
                        v = plsc.load_gather(blk, [fk, c])
                        plsc.store_scatter(group_v, [fk, j], v)

            fire(0, 0)
            fire(1, 1)
            drain(0)
            extract(0, 0)
            fire(2, 0)
            drain(1)
            extract(1, 1)
            fire(3, 1)
            drain(0)
            extract(2, 0)
            drain(1)
            extract(3, 1)
            return 0

        lax.fori_loop(0, _GRP // 16, quad, 0)
        pltpu.sync_copy(group_v, out_t.at[:, pl.ds(base + g0, _GRP)])


def kernel(features, nodes):
    out_t = _gather_t(features.T, nodes)
    return out_t.T


# speedup vs baseline: 2.7087x; 1.1508x over previous
import functools

import jax
import jax.numpy as jnp
from jax import lax
from jax.experimental import pallas as pl
from jax.experimental.pallas import tpu as pltpu
from jax.experimental.pallas import tpu_sc as plsc

DIM = 64
BATCH = 16384

_NC, _NS = 2, 16
_NW = _NC * _NS
_BPW = BATCH // _NW
_GRP = 128
_NGRP = _BPW // _GRP
_NQ = 16


@functools.partial(
    pl.kernel,
    out_type=jax.ShapeDtypeStruct((DIM, BATCH), jnp.float32),
    mesh=plsc.VectorSubcoreMesh(core_axis_name="c", subcore_axis_name="s"),
    scratch_types=[
        pltpu.VMEM((_BPW + 16,), jnp.int32),
        pltpu.VMEM((4, 4, DIM // 2, 128), jnp.float32),
        pltpu.VMEM((DIM, _GRP), jnp.float32),
        pltpu.SemaphoreType.DMA,
        pltpu.SemaphoreType.DMA,
        pltpu.SemaphoreType.DMA,
        pltpu.SemaphoreType.DMA,
    ],
    compiler_params=pltpu.CompilerParams(needs_layout_passes=False),
)
def _gather_t(table_t, nodes_hbm, out_t, nodes_s, blocks_v, group_v, *sems):
    wid = lax.axis_index("s") * _NC + lax.axis_index("c")
    base = wid * _BPW
    pltpu.sync_copy(nodes_hbm.at[pl.ds(base, _BPW)], nodes_s.at[pl.ds(0, _BPW)])
    iota16 = lax.iota(jnp.int32, 16)

    def fire_batch(ndv, b):
        for r in range(4):
            n = ndv[2 * b + r // 2]
            h = r % 2
            col = pl.multiple_of((n >> 7) * 128, 128)
            pltpu.async_copy(
                table_t.at[pl.ds(h * 32, 32), pl.ds(col, 128)],
                blocks_v.at[b, r],
                sems[b],
            )

    def drain_batch(b):
        for r in range(4):
            pltpu.make_async_copy(
                table_t.at[pl.ds(0, 32), pl.ds(0, 128)],
                blocks_v.at[b, r],
                sems[b],
            ).wait()

    def extract_batch(ndv, q, b):
        for r in range(4):
            n = ndv[2 * b + r // 2]
            h = r % 2
            c = jnp.full((16,), n & 127, dtype=jnp.int32)
            j = jnp.full((16,), q * 8 + 2 * b + r // 2, dtype=jnp.int32)
            blk = blocks_v.at[b, r]
            for k in range(2):
                fk = iota16 + (k * 16)
                v = plsc.load_gather(blk, [fk, c])
                plsc.store_scatter(group_v, [fk + h * 32, j], v)

    for g in range(_NGRP):
        g0 = g * _GRP
        ndv0 = nodes_s[pl.ds(g0, 16)]
        for b in range(4):
            fire_batch(ndv0, b)

        def body(q, ndv_q):
            ndv_next = nodes_s[pl.ds(g0 + 8 * (q + 1), 16)]
            for b in range(4):
                drain_batch(b)
                extract_batch(ndv_q, q, b)
                fire_batch(ndv_next, b)
            return ndv_next

        ndv_last = lax.fori_loop(0, _NQ - 1, body, ndv0)
        for b in range(4):
            drain_batch(b)
            extract_batch(ndv_last, _NQ - 1, b)
        pltpu.sync_copy(group_v, out_t.at[:, pl.ds(base + g0, _GRP)])


def kernel(features, nodes):
    out_t = _gather_t(features.T, nodes)
    return out_t.T


# speedup vs baseline: 2.9023x; 1.0715x over previous
import functools

import jax
import jax.numpy as jnp
from jax import lax
from jax.experimental import pallas as pl
from jax.experimental.pallas import tpu as pltpu
from jax.experimental.pallas import tpu_sc as plsc

DIM = 64
BATCH = 16384

_NC, _NS = 2, 16
_NW = _NC * _NS
_BPW = BATCH // _NW
_GRP = 128
_NGRP = _BPW // _GRP
_NQ = _GRP // 16
_NSLOT = 8


@functools.partial(
    pl.kernel,
    out_type=jax.ShapeDtypeStruct((DIM, BATCH), jnp.float32),
    mesh=plsc.VectorSubcoreMesh(core_axis_name="c", subcore_axis_name="s"),
    scratch_types=[
        pltpu.VMEM((_BPW + 16,), jnp.int32),
        pltpu.VMEM((_NSLOT, 4, DIM // 4, 128), jnp.float32),
        pltpu.VMEM((DIM, _GRP), jnp.float32),
    ]
    + [pltpu.SemaphoreType.DMA] * _NSLOT,
    compiler_params=pltpu.CompilerParams(needs_layout_passes=False),
)
def _gather_t(table_t, nodes_hbm, out_t, nodes_s, blocks_v, group_v, *sems):
    wid = lax.axis_index("s") * _NC + lax.axis_index("c")
    base = wid * _BPW
    pltpu.sync_copy(nodes_hbm.at[pl.ds(base, _BPW)], nodes_s.at[pl.ds(0, _BPW)])
    iota16 = lax.iota(jnp.int32, 16)

    def fire(n, slot):
        col = pl.multiple_of((n >> 7) * 128, 128)
        for h in range(4):
            pltpu.async_copy(
                table_t.at[pl.ds(h * 16, 16), pl.ds(col, 128)],
                blocks_v.at[slot, h],
                sems[slot],
            )

    def drain(slot):
        for h in range(4):
            pltpu.make_async_copy(
                table_t.at[pl.ds(0, 16), pl.ds(0, 128)],
                blocks_v.at[slot, h],
                sems[slot],
            ).wait()

    def extract(n, slot, j):
        c = jnp.full((16,), n & 127, dtype=jnp.int32)
        jv = jnp.full((16,), j, dtype=jnp.int32)
        for h in range(4):
            v = plsc.load_gather(blocks_v.at[slot, h], [iota16, c])
            plsc.store_scatter(group_v, [iota16 + h * 16, jv], v)

    for g in range(_NGRP):
        g0 = g * _GRP
        ndv0 = nodes_s[pl.ds(g0, 16)]
        for bb in range(_NSLOT):
            fire(ndv0[bb], bb)

        def body(q, ndv_q):
            ndv_next = nodes_s[pl.ds(g0 + 16 * (q + 1), 16)]
            for bb in range(8):
                drain(bb)
                extract(ndv_q[bb], bb, q * 16 + bb)
                fire(ndv_q[bb + 8], bb)
            for bb in range(8, 16):
                drain(bb - 8)
                extract(ndv_q[bb], bb - 8, q * 16 + bb)
                fire(ndv_next[bb - 8], bb - 8)
            return ndv_next

        ndv_last = lax.fori_loop(0, _NQ - 1, body, ndv0)
        qL = _NQ - 1
        for bb in range(8):
            drain(bb)
            extract(ndv_last[bb], bb, qL * 16 + bb)
            fire(ndv_last[bb + 8], bb)
        for bb in range(8, 16):
            drain(bb - 8)
            extract(ndv_last[bb], bb - 8, qL * 16 + bb)
        pltpu.sync_copy(group_v, out_t.at[:, pl.ds(base + g0, _GRP)])


def kernel(features, nodes):
    out_t = _gather_t(features.T, nodes)
    return out_t.T


# speedup vs baseline: 3.1269x; 1.0774x over previous
import functools

import jax
import jax.numpy as jnp
from jax import lax
from jax.experimental import pallas as pl
from jax.experimental.pallas import tpu as pltpu
from jax.experimental.pallas import tpu_sc as plsc

VOCAB_ = 1000000
DIM = 64
BATCH = 16384

_NC, _NS = 2, 16
_NW = _NC * _NS
_NBLK = (VOCAB_ + 127) // 128
_OWN = (_NBLK + _NW - 1) // _NW
_NSLOT = 3
_TRASH = BATCH
_EMAX = BATCH + 16


@functools.partial(
    pl.kernel,
    out_type=jax.ShapeDtypeStruct((BATCH + 8, 128), jnp.float32),
    mesh=plsc.VectorSubcoreMesh(core_axis_name="c", subcore_axis_name="s"),
    scratch_types=[
        pltpu.VMEM((_EMAX,), jnp.int32),
        pltpu.VMEM((_EMAX,), jnp.int32),
        pltpu.VMEM((_EMAX,), jnp.int32),
        pltpu.VMEM((_EMAX,), jnp.int32),
        pltpu.VMEM((272,), jnp.int32),
        pltpu.VMEM((288,), jnp.int32),
        pltpu.VMEM((32,), jnp.int32),
        pltpu.VMEM((2, _NSLOT, 4, 16, 128), jnp.float32),
        pltpu.VMEM((64, 128), jnp.float32),
        pltpu.VMEM((8, 64), jnp.int32),
    ]
    + [pltpu.SemaphoreType.DMA] * (2 * _NSLOT),
    compiler_params=pltpu.CompilerParams(needs_layout_passes=False),
)
def _gather_d(table_t, nodes_hbm, outp, nodes_a, nn2, ii, nn, ublk, starts,
              bounce, blocks_v, staging_v, idxrow, *sems):
    wid = lax.axis_index("s") * _NC + lax.axis_index("c")
    lo = wid * _OWN
    hi = lo + _OWN
    iota16 = lax.iota(jnp.int32, 16)
    lane0 = iota16 == 0
    pltpu.sync_copy(nodes_hbm, nodes_a.at[pl.ds(0, BATCH)])

    def popcnt(m):
        return plsc.all_reduce_population_count(m)[0]

    def scan_body(t, cnt):
        vn = nodes_a[pl.ds(t * 16, 16)]
        b = vn >> 7
        m = (b >= lo) & (b < hi)
        vi = iota16 + t * 16
        plsc.store_compressed(ii.at[pl.ds(cnt, 16)], vi, mask=m)
        plsc.store_compressed(nn.at[pl.ds(cnt, 16)], vn, mask=m)
        return cnt + popcnt(m)

    e_cnt = lax.fori_loop(0, BATCH // 16, scan_body, 0)
    n_t16 = (e_cnt + 15) >> 4

    def radix_pass(src_i, src_n, dst_i, dst_n, shift):
        cnt2 = 0
        for d in range(16):
            def body(t, cnt, d=d):
                vn = src_n[pl.ds(t * 16, 16)]
                vi = src_i[pl.ds(t * 16, 16)]
                dig = (((vn >> 7) - lo) >> shift) & 15
                valid = (t * 16 + iota16) < e_cnt
                m = (dig == d) & valid
                plsc.store_compressed(dst_n.at[pl.ds(cnt, 16)], vn, mask=m)
                plsc.store_compressed(dst_i.at[pl.ds(cnt, 16)], vi, mask=m)
                return cnt + popcnt(m)

            cnt2 = lax.fori_loop(0, n_t16, body, cnt2)

    radix_pass(ii, nn, nodes_a, nn2, 0)
    radix_pass(nodes_a, nn2, ii, nn, 4)

    for z in range(272 // 16):
        ublk[pl.ds(z * 16, 16)] = jnp.zeros((16,), jnp.int32)
    for z in range(288 // 16):
        starts[pl.ds(z * 16, 16)] = jnp.full((16,), e_cnt, dtype=jnp.int32)

    def uniq_body(t, carry):
        nu, prevb = carry
        vn = nn[pl.ds(t * 16, 16)]
        b = vn >> 7
        plsc.store_compressed(bounce.at[pl.ds(0, 16)],
                              jnp.full((16,), prevb, dtype=jnp.int32),
                              mask=lane0)
        bounce[pl.ds(1, 16)] = b
        sh = bounce[pl.ds(0, 16)]
        valid = (t * 16 + iota16) < e_cnt
        newf = (b != sh) & valid
        plsc.store_compressed(ublk.at[pl.ds(nu, 16)], b, mask=newf)
        plsc.store_compressed(starts.at[pl.ds(nu, 16)], iota16 + t * 16,
                              mask=newf)
        return nu + popcnt(newf), b[15]

    nu, _ = lax.fori_loop(0, n_t16, uniq_body, (0, -1))
    plsc.store_compressed(starts.at[pl.ds(nu, 16)],
                          jnp.full((16,), e_cnt, dtype=jnp.int32), mask=lane0)
    for z in range(4):
        idxrow[0, pl.ds(z * 16, 16)] = jnp.full((16,), _TRASH, dtype=jnp.int32)

    def fetch_sb(s, par):
        for r in range(_NSLOT):
            k = jnp.minimum(s * _NSLOT + r, jnp.maximum(nu - 1, 0))
            bid = ublk[pl.ds(k, 16)][0]
            col = pl.multiple_of(bid * 128, 128)
            for q in range(4):
                pltpu.async_copy(
                    table_t.at[pl.ds(q * 16, 16), pl.ds(col, 128)],
                    blocks_v.at[par, r, q],
                    sems[par * _NSLOT + r],
                )

    def drain_sb(par):
        for r in range(_NSLOT):
            for q in range(4):
                pltpu.make_async_copy(
                    table_t.at[pl.ds(0, 16), pl.ds(0, 128)],
                    blocks_v.at[par, r, q],
                    sems[par * _NSLOT + r],
                ).wait()

    def extract_sb(s, par):
        for r in range(_NSLOT):
            k = s * _NSLOT + r
            s0 = starts[pl.ds(k, 16)][0]
            s1 = starts[pl.ds(k + 1, 16)][0]

            def ent(e, _, r=r, par=par):
                vn = nn[pl.ds(e, 16)][0]
                vi = ii[pl.ds(e, 16)][0]
                c = jnp.full((16,), vn & 127, dtype=jnp.int32)
                slot = e & 63
                for q in range(4):
                    v = plsc.load_gather(blocks_v.at[par, r, q], [iota16, c])
                    staging_v[slot, pl.ds(q * 16, 16)] = v
                plsc.store_scatter(
                    idxrow,
                    [jnp.zeros((16,), jnp.int32),
                     jnp.full((16,), slot, dtype=jnp.int32)],
                    jnp.full((16,), vi, dtype=jnp.int32),
                    mask=lane0,
                )

                def flush():
                    pltpu.sync_copy(staging_v, outp.at[idxrow.at[0]])
                    for z in range(4):
                        idxrow[0, pl.ds(z * 16, 16)] = jnp.full(
                            (16,), _TRASH, dtype=jnp.int32)
                    return 0

                lax.cond(slot == 63, flush, lambda: 0)
                return 0

            lax.fori_loop(s0, s1, ent, 0)

    n_sb = (nu + _NSLOT - 1) // _NSLOT
    n_pair = (n_sb + 1) >> 1
    fetch_sb(0, 0)

    def pair_body(p, _):
        s = p * 2
        fetch_sb(s + 1, 1)
        drain_sb(0)
        extract_sb(s, 0)
        fetch_sb(s + 2, 0)
        drain_sb(1)
        extract_sb(s + 1, 1)
        return 0

    lax.fori_loop(0, n_pair, pair_body, 0)
    drain_sb(0)
    extract_sb(2 * n_pair, 0)

    def final_flush():
        pltpu.sync_copy(staging_v, outp.at[idxrow.at[0]])
        return 0

    lax.cond((e_cnt & 63) != 0, final_flush, lambda: 0)


def kernel(features, nodes):
    outp = _gather_d(features.T, nodes)
    return outp[:BATCH, :DIM]


# speedup vs baseline: 3.1270x; 1.0000x over previous
import functools

import jax
import jax.numpy as jnp
from jax import lax
from jax.experimental import pallas as pl
from jax.experimental.pallas import tpu as pltpu
from jax.experimental.pallas import tpu_sc as plsc

VOCAB_ = 1000000
DIM = 64
BATCH = 16384

_NC, _NS = 2, 16
_NW = _NC * _NS
_NBLK = (VOCAB_ + 127) // 128
_OWN = (_NBLK + _NW - 1) // _NW
_NSLOT = 3
_TRASH = BATCH
_EMAX = BATCH + 16


@functools.partial(
    pl.kernel,
    out_type=jax.ShapeDtypeStruct((BATCH + 8, 128), jnp.float32),
    mesh=plsc.VectorSubcoreMesh(core_axis_name="c", subcore_axis_name="s"),
    scratch_types=[
        pltpu.VMEM((_EMAX,), jnp.int32),
        pltpu.VMEM((_EMAX,), jnp.int32),
        pltpu.VMEM((_EMAX,), jnp.int32),
        pltpu.VMEM((_EMAX,), jnp.int32),
        pltpu.VMEM((272,), jnp.int32),
        pltpu.VMEM((288,), jnp.int32),
        pltpu.VMEM((32,), jnp.int32),
        pltpu.VMEM((2, _NSLOT, 64, 128), jnp.float32),
        pltpu.VMEM((64, 128), jnp.float32),
        pltpu.VMEM((8, 64), jnp.int32),
    ]
    + [pltpu.SemaphoreType.DMA] * (2 * _NSLOT),
    compiler_params=pltpu.CompilerParams(needs_layout_passes=False),
)
def _gather_d(table_t, nodes_hbm, outp, nodes_a, nn2, ii, nn, ublk, starts,
              bounce, blocks_v, staging_v, idxrow, *sems):
    wid = lax.axis_index("s") * _NC + lax.axis_index("c")
    lo = wid * _OWN
    hi = lo + _OWN
    iota16 = lax.iota(jnp.int32, 16)
    lane0 = iota16 == 0
    pltpu.sync_copy(nodes_hbm, nodes_a.at[pl.ds(0, BATCH)])

    def popcnt(m):
        return plsc.all_reduce_population_count(m)[0]

    def scan_body(t, cnt):
        vn = nodes_a[pl.ds(t * 16, 16)]
        b = vn >> 7
        m = (b >= lo) & (b < hi)
        vi = iota16 + t * 16
        plsc.store_compressed(ii.at[pl.ds(cnt, 16)], vi, mask=m)
        plsc.store_compressed(nn.at[pl.ds(cnt, 16)], vn, mask=m)
        return cnt + popcnt(m)

    e_cnt = lax.fori_loop(0, BATCH // 16, scan_body, 0)
    n_t16 = (e_cnt + 15) >> 4

    def radix_pass(src_i, src_n, dst_i, dst_n, shift):
        cnt2 = 0
        for d in range(16):
            def body(t, cnt, d=d):
                vn = src_n[pl.ds(t * 16, 16)]
                vi = src_i[pl.ds(t * 16, 16)]
                dig = (((vn >> 7) - lo) >> shift) & 15
                valid = (t * 16 + iota16) < e_cnt
                m = (dig == d) & valid
                plsc.store_compressed(dst_n.at[pl.ds(cnt, 16)], vn, mask=m)
                plsc.store_compressed(dst_i.at[pl.ds(cnt, 16)], vi, mask=m)
                return cnt + popcnt(m)

            cnt2 = lax.fori_loop(0, n_t16, body, cnt2)

    radix_pass(ii, nn, nodes_a, nn2, 0)
    radix_pass(nodes_a, nn2, ii, nn, 4)

    for z in range(272 // 16):
        ublk[pl.ds(z * 16, 16)] = jnp.zeros((16,), jnp.int32)
    for z in range(288 // 16):
        starts[pl.ds(z * 16, 16)] = jnp.full((16,), e_cnt, dtype=jnp.int32)

    def uniq_body(t, carry):
        nu, prevb = carry
        vn = nn[pl.ds(t * 16, 16)]
        b = vn >> 7
        plsc.store_compressed(bounce.at[pl.ds(0, 16)],
                              jnp.full((16,), prevb, dtype=jnp.int32),
                              mask=lane0)
        bounce[pl.ds(1, 16)] = b
        sh = bounce[pl.ds(0, 16)]
        valid = (t * 16 + iota16) < e_cnt
        newf = (b != sh) & valid
        plsc.store_compressed(ublk.at[pl.ds(nu, 16)], b, mask=newf)
        plsc.store_compressed(starts.at[pl.ds(nu, 16)], iota16 + t * 16,
                              mask=newf)
        return nu + popcnt(newf), b[15]

    nu, _ = lax.fori_loop(0, n_t16, uniq_body, (0, -1))
    plsc.store_compressed(starts.at[pl.ds(nu, 16)],
                          jnp.full((16,), e_cnt, dtype=jnp.int32), mask=lane0)
    for z in range(4):
        idxrow[0, pl.ds(z * 16, 16)] = jnp.full((16,), _TRASH, dtype=jnp.int32)

    def fetch_sb(s, par):
        for r in range(_NSLOT):
            k = jnp.minimum(s * _NSLOT + r, jnp.maximum(nu - 1, 0))
            bid = ublk[pl.ds(k, 16)][0]
            col = pl.multiple_of(bid * 128, 128)
            pltpu.async_copy(
                table_t.at[pl.ds(0, 64), pl.ds(col, 128)],
                blocks_v.at[par, r],
                sems[par * _NSLOT + r],
            )

    def drain_sb(par):
        for r in range(_NSLOT):
            pltpu.make_async_copy(
                table_t.at[pl.ds(0, 64), pl.ds(0, 128)],
                blocks_v.at[par, r],
                sems[par * _NSLOT + r],
            ).wait()

    def extract_sb(s, par):
        for r in range(_NSLOT):
            k = s * _NSLOT + r
            s0 = starts[pl.ds(k, 16)][0]
            s1 = starts[pl.ds(k + 1, 16)][0]

            def ent(e, _, r=r, par=par):
                vn = nn[pl.ds(e, 16)][0]
                vi = ii[pl.ds(e, 16)][0]
                c = jnp.full((16,), vn & 127, dtype=jnp.int32)
                slot = e & 63
                for q in range(4):
                    v = plsc.load_gather(blocks_v.at[par, r],
                                         [iota16 + q * 16, c])
                    staging_v[slot, pl.ds(q * 16, 16)] = v
                plsc.store_scatter(
                    idxrow,
                    [jnp.zeros((16,), jnp.int32),
                     jnp.full((16,), slot, dtype=jnp.int32)],
                    jnp.full((16,), vi, dtype=jnp.int32),
                    mask=lane0,
                )

                def flush():
                    pltpu.sync_copy(staging_v, outp.at[idxrow.at[0]])
                    for z in range(4):
                        idxrow[0, pl.ds(z * 16, 16)] = jnp.full(
                            (16,), _TRASH, dtype=jnp.int32)
                    return 0

                lax.cond(slot == 63, flush, lambda: 0)
                return 0

            lax.fori_loop(s0, s1, ent, 0)

    n_sb = (nu + _NSLOT - 1) // _NSLOT
    n_pair = (n_sb + 1) >> 1
    fetch_sb(0, 0)

    def pair_body(p, _):
        s = p * 2
        fetch_sb(s + 1, 1)
        drain_sb(0)
        extract_sb(s, 0)
        fetch_sb(s + 2, 0)
        drain_sb(1)
        extract_sb(s + 1, 1)
        return 0

    lax.fori_loop(0, n_pair, pair_body, 0)
    drain_sb(0)
    extract_sb(2 * n_pair, 0)

    def final_flush():
        pltpu.sync_copy(staging_v, outp.at[idxrow.at[0]])
        return 0

    lax.cond((e_cnt & 63) != 0, final_flush, lambda: 0)


def kernel(features, nodes):
    outp = _gather_d(features.T, nodes)
    return outp[:BATCH, :DIM]
